# baseline (speedup 1.0000x reference)
import functools

import jax
import jax.numpy as jnp
from jax import lax
from jax.experimental import pallas as pl
from jax.experimental.pallas import tpu as pltpu
from jax.experimental.pallas import tpu_sc as plsc

N = 4096
E = 262144
D_IN = 256
D_EDGE = 4
D_OUT = 256

NC = 2
NS = 16
NW = NC * NS
EPW = E // NW
K = 128
NCHUNK = EPW // K
RPT = N // NS

_mesh = plsc.VectorSubcoreMesh(core_axis_name="c", subcore_axis_name="s")


def _h_body(x_ref, w_ref, b_ref, h_ref):
    h_ref[...] = (
        jnp.dot(x_ref[...], w_ref[...], preferred_element_type=jnp.float32)
        + b_ref[...]
    )


EAROW = 16


def _gather_body(h_hbm, row_hbm, col_hbm, ea_hbm, g0_hbm, eg_hbm,
                 ridx_v, cidx_v, eidx_v, qidx_v, hrows_v, erows_v, eg_v,
                 sem_h, sem_e):
    c = lax.axis_index("c")
    s = lax.axis_index("s")
    wid = s * NC + c
    base0 = wid * EPW
    lane = lax.iota(jnp.int32, 16)
    lrow = lax.shift_right_logical(lane, 2)
    lcomp = lane & 3

    def body(j, carry):
        base = base0 + j * K
        pltpu.sync_copy(row_hbm.at[pl.ds(base, K)], ridx_v)
        pltpu.sync_copy(col_hbm.at[pl.ds(base, K)], cidx_v)
        for i in range(K // 16):
            sl = pl.ds(i * 16, 16)
            fidx = ridx_v[sl] * N + cidx_v[sl]
            eidx_v[sl] = fidx
            qidx_v[sl] = lax.shift_right_logical(fidx, 2)
        cp_h = pltpu.async_copy(h_hbm.at[ridx_v], hrows_v, sem_h)
        cp_e = pltpu.async_copy(ea_hbm.at[qidx_v], erows_v, sem_e)
        cp_h.wait()
        cp_e.wait()
        def extract(g, carry2):
            trow = g * 4 + lrow
            fidx = plsc.load_gather(eidx_v, [trow])
            off = lax.shift_left((fidx & 3), 2) + lcomp
            vals = plsc.load_gather(erows_v, [trow, off])
            eg_v[pl.ds(g * 16, 16)] = vals
            return carry2
        lax.fori_loop(0, K * 4 // 16, extract, 0)
        pltpu.sync_copy(hrows_v, g0_hbm.at[pl.ds(base, K)])
        pltpu.sync_copy(eg_v, eg_hbm.at[pl.ds(base * 4, K * 4)])
        return carry

    lax.fori_loop(0, NCHUNK, body, 0)


@functools.partial(
    pl.kernel,
    out_type=(
        jax.ShapeDtypeStruct((E, D_OUT), jnp.float32),
        jax.ShapeDtypeStruct((E * D_EDGE,), jnp.float32),
    ),
    mesh=_mesh,
    scratch_types=[
        pltpu.VMEM((K,), jnp.int32),
        pltpu.VMEM((K,), jnp.int32),
        pltpu.VMEM((K,), jnp.int32),
        pltpu.VMEM((K,), jnp.int32),
        pltpu.VMEM((K, D_OUT), jnp.float32),
        pltpu.VMEM((K, EAROW), jnp.float32),
        pltpu.VMEM((K * D_EDGE,), jnp.float32),
        pltpu.SemaphoreType.DMA,
        pltpu.SemaphoreType.DMA,
    ],
    compiler_params=pltpu.CompilerParams(
        use_tc_tiling_on_sc=False, needs_layout_passes=False),
)
def _sc_gather(h_hbm, row_hbm, col_hbm, ea_hbm, g0_hbm, eg_hbm,
               ridx_v, cidx_v, eidx_v, qidx_v, hrows_v, erows_v, eg_v,
               sem_h, sem_e):
    _gather_body(h_hbm, row_hbm, col_hbm, ea_hbm, g0_hbm, eg_hbm,
                 ridx_v, cidx_v, eidx_v, qidx_v, hrows_v, erows_v, eg_v,
                 sem_h, sem_e)


def _mlp_body(g0_ref, eg_ref, w1e_ref, w2_ref, b2_ref, out_ref):
    ew = jnp.dot(eg_ref[...], w1e_ref[...], preferred_element_type=jnp.float32)
    h1 = jnp.maximum(g0_ref[...] + ew, 0.0)
    h2 = (
        jnp.dot(h1, w2_ref[...], preferred_element_type=jnp.float32)
        + b2_ref[...]
    )
    out_ref[...] = jnp.maximum(h2, 0.0)


@functools.partial(
    pl.kernel,
    out_type=jax.ShapeDtypeStruct((NC * N, D_OUT), jnp.float32),
    mesh=_mesh,
    scratch_types=[
        pltpu.VMEM((K,), jnp.int32),
        pltpu.VMEM((K, D_OUT), jnp.float32),
        pltpu.VMEM_SHARED((N, D_OUT), jnp.float32),
    ],
    compiler_params=pltpu.CompilerParams(use_tc_tiling_on_sc=False),
)
def _sc_scatter(out2_hbm, col_hbm, zeros_hbm, part_hbm, cidx_v, rows_v, acc_sh):
    c = lax.axis_index("c")
    s = lax.axis_index("s")
    wid = s * NC + c
    base0 = wid * EPW

    pltpu.sync_copy(zeros_hbm.at[pl.ds(s * RPT, RPT)],
                    acc_sh.at[pl.ds(s * RPT, RPT)])
    plsc.subcore_barrier()

    def body(j, carry):
        base = base0 + j * K
        pltpu.sync_copy(col_hbm.at[pl.ds(base, K)], cidx_v)
        pltpu.sync_copy(out2_hbm.at[pl.ds(base, K)], rows_v)
        pltpu.sync_copy(rows_v, acc_sh.at[cidx_v], add=True)
        return carry

    lax.fori_loop(0, NCHUNK, body, 0)
    plsc.subcore_barrier()

    pltpu.sync_copy(acc_sh.at[pl.ds(s * RPT, RPT)],
                    part_hbm.at[pl.ds(c * N + s * RPT, RPT)])


def _combine_body(p_ref, o_ref):
    o_ref[...] = p_ref[0] + p_ref[1]


def kernel(x, edge_index, edge_attr, W1, b1, W2, b2):
    row = edge_index[0]
    col = edge_index[1]
    W1xT = W1[:, :D_IN].T
    W1eT = W1[:, D_IN:].T
    W2T = W2.T
    ea_flat = edge_attr.reshape(N * N * D_EDGE // EAROW, EAROW)

    H = pl.pallas_call(
        _h_body,
        out_shape=jax.ShapeDtypeStruct((N, D_OUT), jnp.float32),
    )(x, W1xT, b1.reshape(1, D_OUT))

    g0, eg = _sc_gather(H, row, col, ea_flat)
    eg = eg.reshape(E, D_EDGE)

    B = 1024
    out2 = pl.pallas_call(
        _mlp_body,
        grid=(E // B,),
        in_specs=[
            pl.BlockSpec((B, D_OUT), lambda i: (i, 0)),
            pl.BlockSpec((B, D_EDGE), lambda i: (i, 0)),
            pl.BlockSpec((D_EDGE, D_OUT), lambda i: (0, 0)),
            pl.BlockSpec((D_OUT, D_OUT), lambda i: (0, 0)),
            pl.BlockSpec((1, D_OUT), lambda i: (0, 0)),
        ],
        out_specs=pl.BlockSpec((B, D_OUT), lambda i: (i, 0)),
        out_shape=jax.ShapeDtypeStruct((E, D_OUT), jnp.float32),
    )(g0, eg, W1eT, W2T, b2.reshape(1, D_OUT))

    zeros = jnp.zeros((N, D_OUT), jnp.float32)
    part = _sc_scatter(out2, col, zeros)

    BR = 512
    out = pl.pallas_call(
        _combine_body,
        grid=(N // BR,),
        in_specs=[pl.BlockSpec((NC, BR, D_OUT), lambda i: (0, i, 0))],
        out_specs=pl.BlockSpec((BR, D_OUT), lambda i: (i, 0)),
        out_shape=jax.ShapeDtypeStruct((N, D_OUT), jnp.float32),
    )(part.reshape(NC, N, D_OUT))
    return out


# speedup vs baseline: 17.1346x; 17.1346x over previous
import functools

import jax
import jax.numpy as jnp
from jax import lax
from jax.experimental import pallas as pl
from jax.experimental.pallas import tpu as pltpu
from jax.experimental.pallas import tpu_sc as plsc

N = 4096
E = 262144
D_IN = 256
D_EDGE = 4
D_OUT = 256

NC = 2
NS = 16
NW = NC * NS
EPW = E // NW
K = 128
NCHUNK = EPW // K
RPT = N // NS

_mesh = plsc.VectorSubcoreMesh(core_axis_name="c", subcore_axis_name="s")


def _h_body(x_ref, w_ref, b_ref, h_ref):
    h_ref[...] = (
        jnp.dot(x_ref[...], w_ref[...], preferred_element_type=jnp.float32)
        + b_ref[...]
    )


EAROW = 16


def _gather_body(h_hbm, row_hbm, col_hbm, ea_hbm, g0_hbm, eg_hbm,
                 ridx_v, cidx_v, qidx_v, hrows_v, erows_v, eg_v,
                 sem_h, sem_e):
    c = lax.axis_index("c")
    s = lax.axis_index("s")
    wid = s * NC + c
    base0 = wid * EPW
    lane = lax.iota(jnp.int32, 16)

    def body(j, carry):
        base = base0 + j * K
        pltpu.sync_copy(row_hbm.at[pl.ds(base, K)], ridx_v)
        pltpu.sync_copy(col_hbm.at[pl.ds(base, K)], cidx_v)
        for i in range(K // 16):
            sl = pl.ds(i * 16, 16)
            r = ridx_v[sl]
            cc = cidx_v[sl]
            bq = (r << 10) + ((cc >> 7) << 5) + ((cc >> 4) & 7)
            for f in range(D_EDGE):
                qidx_v[f, sl] = bq + f * 8
        cp_h = pltpu.async_copy(h_hbm.at[ridx_v], hrows_v, sem_h)
        cps = [pltpu.async_copy(ea_hbm.at[qidx_v.at[f]], erows_v.at[f], sem_e)
               for f in range(D_EDGE)]
        cp_h.wait()
        for cp in cps:
            cp.wait()
        for f in range(D_EDGE):
            fvec = jnp.full((16,), f, jnp.int32)
            for g in range(K // 16):
                sl = pl.ds(g * 16, 16)
                trow = lane + g * 16
                ccol = cidx_v[sl] & 15
                vals = plsc.load_gather(erows_v, [fvec, trow, ccol])
                eg_v[f, pl.ds(j * K + g * 16, 16)] = vals
        pltpu.sync_copy(hrows_v, g0_hbm.at[pl.ds(base, K)])
        return carry

    lax.fori_loop(0, NCHUNK, body, 0)
    pltpu.sync_copy(eg_v, eg_hbm.at[wid])


@functools.partial(
    pl.kernel,
    out_type=(
        jax.ShapeDtypeStruct((E, D_OUT), jnp.float32),
        jax.ShapeDtypeStruct((NW, D_EDGE, EPW), jnp.float32),
    ),
    mesh=_mesh,
    scratch_types=[
        pltpu.VMEM((K,), jnp.int32),
        pltpu.VMEM((K,), jnp.int32),
        pltpu.VMEM((D_EDGE, K), jnp.int32),
        pltpu.VMEM((K, D_OUT), jnp.float32),
        pltpu.VMEM((D_EDGE, K, EAROW), jnp.float32),
        pltpu.VMEM((D_EDGE, EPW), jnp.float32),
        pltpu.SemaphoreType.DMA,
        pltpu.SemaphoreType.DMA,
    ],
    compiler_params=pltpu.CompilerParams(
        use_tc_tiling_on_sc=False, needs_layout_passes=False),
)
def _sc_gather(h_hbm, row_hbm, col_hbm, ea_hbm, g0_hbm, eg_hbm,
               ridx_v, cidx_v, qidx_v, hrows_v, erows_v, eg_v,
               sem_h, sem_e):
    _gather_body(h_hbm, row_hbm, col_hbm, ea_hbm, g0_hbm, eg_hbm,
                 ridx_v, cidx_v, qidx_v, hrows_v, erows_v, eg_v,
                 sem_h, sem_e)


def _mlp_body(g0_ref, eg_ref, w1e_ref, w2_ref, b2_ref, out_ref):
    egb = eg_ref[0]
    ew = lax.dot_general(egb, w1e_ref[...], (((0,), (0,)), ((), ())),
                         preferred_element_type=jnp.float32)
    h1 = jnp.maximum(g0_ref[...] + ew, 0.0)
    h2 = (
        jnp.dot(h1, w2_ref[...], preferred_element_type=jnp.float32)
        + b2_ref[...]
    )
    out_ref[...] = jnp.maximum(h2, 0.0)


@functools.partial(
    pl.kernel,
    out_type=jax.ShapeDtypeStruct((NC * N, D_OUT), jnp.float32),
    mesh=_mesh,
    scratch_types=[
        pltpu.VMEM((K,), jnp.int32),
        pltpu.VMEM((K, D_OUT), jnp.float32),
        pltpu.VMEM_SHARED((N, D_OUT), jnp.float32),
    ],
    compiler_params=pltpu.CompilerParams(use_tc_tiling_on_sc=False),
)
def _sc_scatter(out2_hbm, col_hbm, zeros_hbm, part_hbm, cidx_v, rows_v, acc_sh):
    c = lax.axis_index("c")
    s = lax.axis_index("s")
    wid = s * NC + c
    base0 = wid * EPW

    pltpu.sync_copy(zeros_hbm.at[pl.ds(s * RPT, RPT)],
                    acc_sh.at[pl.ds(s * RPT, RPT)])
    plsc.subcore_barrier()

    def body(j, carry):
        base = base0 + j * K
        pltpu.sync_copy(col_hbm.at[pl.ds(base, K)], cidx_v)
        pltpu.sync_copy(out2_hbm.at[pl.ds(base, K)], rows_v)
        pltpu.sync_copy(rows_v, acc_sh.at[cidx_v], add=True)
        return carry

    lax.fori_loop(0, NCHUNK, body, 0)
    plsc.subcore_barrier()

    pltpu.sync_copy(acc_sh.at[pl.ds(s * RPT, RPT)],
                    part_hbm.at[pl.ds(c * N + s * RPT, RPT)])


def _combine_body(p_ref, o_ref):
    o_ref[...] = p_ref[0] + p_ref[1]


def kernel(x, edge_index, edge_attr, W1, b1, W2, b2):
    row = edge_index[0]
    col = edge_index[1]
    W1xT = W1[:, :D_IN].T
    W1eT = W1[:, D_IN:].T
    W2T = W2.T
    ea_flat = (edge_attr.reshape(N, N // 128, 128, D_EDGE)
               .transpose(0, 1, 3, 2)
               .reshape(N * N * D_EDGE // EAROW, EAROW))

    H = pl.pallas_call(
        _h_body,
        out_shape=jax.ShapeDtypeStruct((N, D_OUT), jnp.float32),
    )(x, W1xT, b1.reshape(1, D_OUT))

    g0, eg = _sc_gather(H, row, col, ea_flat)

    B = 1024
    BPW = EPW // B
    out2 = pl.pallas_call(
        _mlp_body,
        grid=(E // B,),
        in_specs=[
            pl.BlockSpec((B, D_OUT), lambda i: (i, 0)),
            pl.BlockSpec((1, D_EDGE, B), lambda i: (i // BPW, 0, i % BPW)),
            pl.BlockSpec((D_EDGE, D_OUT), lambda i: (0, 0)),
            pl.BlockSpec((D_OUT, D_OUT), lambda i: (0, 0)),
            pl.BlockSpec((1, D_OUT), lambda i: (0, 0)),
        ],
        out_specs=pl.BlockSpec((B, D_OUT), lambda i: (i, 0)),
        out_shape=jax.ShapeDtypeStruct((E, D_OUT), jnp.float32),
    )(g0, eg, W1eT, W2T, b2.reshape(1, D_OUT))

    zeros = jnp.zeros((N, D_OUT), jnp.float32)
    part = _sc_scatter(out2, col, zeros)

    BR = 512
    out = pl.pallas_call(
        _combine_body,
        grid=(N // BR,),
        in_specs=[pl.BlockSpec((NC, BR, D_OUT), lambda i: (0, i, 0))],
        out_specs=pl.BlockSpec((BR, D_OUT), lambda i: (i, 0)),
        out_shape=jax.ShapeDtypeStruct((N, D_OUT), jnp.float32),
    )(part.reshape(NC, N, D_OUT))
    return out


# speedup vs baseline: 19.5261x; 1.1396x over previous
import functools

import jax
import jax.numpy as jnp
from jax import lax
from jax.experimental import pallas as pl
from jax.experimental.pallas import tpu as pltpu
from jax.experimental.pallas import tpu_sc as plsc

N = 4096
E = 262144
D_IN = 256
D_EDGE = 4
D_OUT = 256

NC = 2
NS = 16
NW = NC * NS
EPW = E // NW
K = 128
NCHUNK = EPW // K
RPT = N // NS

_mesh = plsc.VectorSubcoreMesh(core_axis_name="c", subcore_axis_name="s")


def _h_body(x_ref, w_ref, b_ref, h_ref):
    h_ref[...] = (
        jnp.dot(x_ref[...], w_ref[...], preferred_element_type=jnp.float32)
        + b_ref[...]
    )


EAROW = 16


def _gather_body(h_hbm, row_hbm, col_hbm, ea_hbm, g0_hbm, eg_hbm,
                 ridx_v, cidx_v, qidx_v, hrows_v, erows_v, eg_v,
                 sem_h, sem_e):
    c = lax.axis_index("c")
    s = lax.axis_index("s")
    wid = s * NC + c
    base0 = wid * EPW
    lane = lax.iota(jnp.int32, 16)

    def body(j, carry):
        base = base0 + j * K
        pltpu.sync_copy(row_hbm.at[pl.ds(base, K)], ridx_v)
        pltpu.sync_copy(col_hbm.at[pl.ds(base, K)], cidx_v)
        for i in range(K // 16):
            sl = pl.ds(i * 16, 16)
            r = ridx_v[sl]
            cc = cidx_v[sl]
            bq = (r << 10) + ((cc >> 7) << 5) + ((cc >> 4) & 7)
            for f in range(D_EDGE):
                qidx_v[f, sl] = bq + f * 8
        cp_h = pltpu.async_copy(h_hbm.at[ridx_v], hrows_v, sem_h)
        cps = [pltpu.async_copy(ea_hbm.at[qidx_v.at[f]], erows_v.at[f], sem_e)
               for f in range(D_EDGE)]
        cp_h.wait()
        for cp in cps:
            cp.wait()
        for f in range(D_EDGE):
            fvec = jnp.full((16,), f, jnp.int32)
            for g in range(K // 16):
                sl = pl.ds(g * 16, 16)
                trow = lane + g * 16
                ccol = cidx_v[sl] & 15
                vals = plsc.load_gather(erows_v, [fvec, trow, ccol])
                eg_v[f, pl.ds(j * K + g * 16, 16)] = vals
        pltpu.sync_copy(hrows_v, g0_hbm.at[pl.ds(base, K)])
        return carry

    lax.fori_loop(0, NCHUNK, body, 0)
    pltpu.sync_copy(eg_v, eg_hbm.at[wid])


@functools.partial(
    pl.kernel,
    out_type=(
        jax.ShapeDtypeStruct((E, D_OUT), jnp.float32),
        jax.ShapeDtypeStruct((NW, D_EDGE, EPW), jnp.float32),
    ),
    mesh=_mesh,
    scratch_types=[
        pltpu.VMEM((K,), jnp.int32),
        pltpu.VMEM((K,), jnp.int32),
        pltpu.VMEM((D_EDGE, K), jnp.int32),
        pltpu.VMEM((K, D_OUT), jnp.float32),
        pltpu.VMEM((D_EDGE, K, EAROW), jnp.float32),
        pltpu.VMEM((D_EDGE, EPW), jnp.float32),
        pltpu.SemaphoreType.DMA,
        pltpu.SemaphoreType.DMA,
    ],
    compiler_params=pltpu.CompilerParams(
        use_tc_tiling_on_sc=False, needs_layout_passes=False),
)
def _sc_gather(h_hbm, row_hbm, col_hbm, ea_hbm, g0_hbm, eg_hbm,
               ridx_v, cidx_v, qidx_v, hrows_v, erows_v, eg_v,
               sem_h, sem_e):
    _gather_body(h_hbm, row_hbm, col_hbm, ea_hbm, g0_hbm, eg_hbm,
                 ridx_v, cidx_v, qidx_v, hrows_v, erows_v, eg_v,
                 sem_h, sem_e)


def _mlp_body(g0_ref, eg_ref, w1e_ref, w2_ref, b2_ref, out_ref):
    egb = eg_ref[0]
    ew = lax.dot_general(egb, w1e_ref[...], (((0,), (0,)), ((), ())),
                         preferred_element_type=jnp.float32)
    h1 = jnp.maximum(g0_ref[...] + ew, 0.0)
    h2 = (
        jnp.dot(h1.astype(jnp.bfloat16), w2_ref[...],
                preferred_element_type=jnp.float32)
        + b2_ref[...]
    )
    out_ref[...] = jnp.maximum(h2, 0.0)


HN = 2 * N
HRPT = HN // NS


@functools.partial(
    pl.kernel,
    out_type=jax.ShapeDtypeStruct((NC * HN, 128), jnp.float32),
    mesh=_mesh,
    scratch_types=[
        pltpu.VMEM((K,), jnp.int32),
        pltpu.VMEM((2, K), jnp.int32),
        pltpu.VMEM((2 * K, 128), jnp.float32),
        pltpu.VMEM_SHARED((HN, 128), jnp.float32),
    ],
    compiler_params=pltpu.CompilerParams(
        use_tc_tiling_on_sc=False, needs_layout_passes=False),
)
def _sc_scatter(out2t_hbm, col_hbm, zeros_hbm, part_hbm,
                cidx_v, didx_v, rows_v, acc_sh):
    c = lax.axis_index("c")
    s = lax.axis_index("s")
    wid = s * NC + c
    base0 = wid * EPW
    lane = lax.iota(jnp.int32, 16)
    lane7 = lane & 7
    lhalf = lax.shift_right_logical(lane, 3) & 1

    pltpu.sync_copy(zeros_hbm.at[pl.ds(s * HRPT, HRPT)],
                    acc_sh.at[pl.ds(s * HRPT, HRPT)])
    plsc.subcore_barrier()

    def body(j, carry):
        base = base0 + j * K
        pltpu.sync_copy(col_hbm.at[pl.ds(base, K)], cidx_v)
        pltpu.sync_copy(out2t_hbm.at[pl.ds(2 * base, 2 * K)], rows_v)
        for g in range(2 * K // 16):
            cols = plsc.load_gather(cidx_v, [g * 8 + lane7])
            d = 2 * cols + lhalf
            didx_v[g // 8, pl.ds((g % 8) * 16, 16)] = d
        for q in range(2):
            pltpu.sync_copy(rows_v.at[pl.ds(q * K, K)],
                            acc_sh.at[didx_v.at[q]], add=True)
        return carry

    lax.fori_loop(0, NCHUNK, body, 0)
    plsc.subcore_barrier()

    pltpu.sync_copy(acc_sh.at[pl.ds(s * HRPT, HRPT)],
                    part_hbm.at[pl.ds(c * HN + s * HRPT, HRPT)])


def _combine_body(p_ref, o_ref):
    o_ref[...] = p_ref[0] + p_ref[1]


def kernel(x, edge_index, edge_attr, W1, b1, W2, b2):
    row = edge_index[0]
    col = edge_index[1]
    W1xT = W1[:, :D_IN].T
    W1eT = W1[:, D_IN:].T
    W2T = W2.T
    ea_flat = (edge_attr.reshape(N, N // 128, 128, D_EDGE)
               .transpose(0, 1, 3, 2)
               .reshape(N * N * D_EDGE // EAROW, EAROW))

    H = pl.pallas_call(
        _h_body,
        out_shape=jax.ShapeDtypeStruct((N, D_OUT), jnp.float32),
    )(x, W1xT, b1.reshape(1, D_OUT))

    g0, eg = _sc_gather(H, row, col, ea_flat)

    B = 1024
    BPW = EPW // B
    out2 = pl.pallas_call(
        _mlp_body,
        grid=(E // B,),
        in_specs=[
            pl.BlockSpec((B, D_OUT), lambda i: (i, 0)),
            pl.BlockSpec((1, D_EDGE, B), lambda i: (i // BPW, 0, i % BPW)),
            pl.BlockSpec((D_EDGE, D_OUT), lambda i: (0, 0)),
            pl.BlockSpec((D_OUT, D_OUT), lambda i: (0, 0)),
            pl.BlockSpec((1, D_OUT), lambda i: (0, 0)),
        ],
        out_specs=pl.BlockSpec((B, D_OUT), lambda i: (i, 0)),
        out_shape=jax.ShapeDtypeStruct((E, D_OUT), jnp.float32),
    )(g0, eg, W1eT, W2T.astype(jnp.bfloat16), b2.reshape(1, D_OUT))

    out2t = (out2.reshape(E // 8, 8, 2, 128)
             .transpose(0, 2, 1, 3)
             .reshape(2 * E, 128))
    zeros = jnp.zeros((HN, 128), jnp.float32)
    part = _sc_scatter(out2t, col, zeros)

    BR = 512
    out = pl.pallas_call(
        _combine_body,
        grid=(N // BR,),
        in_specs=[pl.BlockSpec((NC, BR, D_OUT), lambda i: (0, i, 0))],
        out_specs=pl.BlockSpec((BR, D_OUT), lambda i: (i, 0)),
        out_shape=jax.ShapeDtypeStruct((N, D_OUT), jnp.float32),
    )(part.reshape(NC, N, D_OUT))
    return out


# speedup vs baseline: 21.0824x; 1.0797x over previous
import functools

import jax
import jax.numpy as jnp
from jax import lax
from jax.experimental import pallas as pl
from jax.experimental.pallas import tpu as pltpu
from jax.experimental.pallas import tpu_sc as plsc

N = 4096
E = 262144
D_IN = 256
D_EDGE = 4
D_OUT = 256

NC = 2
NS = 16
NW = NC * NS
EPW = E // NW
K = 128
NCHUNK = EPW // K
RPT = N // NS

_mesh = plsc.VectorSubcoreMesh(core_axis_name="c", subcore_axis_name="s")


def _h_body(x_ref, w_ref, b_ref, h_ref):
    h_ref[...] = (
        jnp.dot(x_ref[...], w_ref[...], preferred_element_type=jnp.float32)
        + b_ref[...]
    )


EAROW = 16


def _gather_body(h_hbm, row_hbm, col_hbm, ea_hbm, g0_hbm, eg_hbm,
                 ridx_v, cidx_v, qidx_v, hrows_v, erows_v, eg_v,
                 sem_h, sem_e):
    c = lax.axis_index("c")
    s = lax.axis_index("s")
    wid = s * NC + c
    base0 = wid * EPW
    lane = lax.iota(jnp.int32, 16)

    def body(j, carry):
        base = base0 + j * K
        pltpu.sync_copy(row_hbm.at[pl.ds(base, K)], ridx_v)
        pltpu.sync_copy(col_hbm.at[pl.ds(base, K)], cidx_v)
        for i in range(K // 16):
            sl = pl.ds(i * 16, 16)
            r = ridx_v[sl]
            cc = cidx_v[sl]
            bq = (r << 10) + ((cc >> 7) << 5) + ((cc >> 4) & 7)
            for f in range(D_EDGE):
                qidx_v[f, sl] = bq + f * 8
        cp_h = pltpu.async_copy(h_hbm.at[ridx_v], hrows_v, sem_h)
        cps = [pltpu.async_copy(ea_hbm.at[qidx_v.at[f]], erows_v.at[f], sem_e)
               for f in range(D_EDGE)]
        cp_h.wait()
        for cp in cps:
            cp.wait()
        for f in range(D_EDGE):
            fvec = jnp.full((16,), f, jnp.int32)
            for g in range(K // 16):
                sl = pl.ds(g * 16, 16)
                trow = lane + g * 16
                ccol = cidx_v[sl] & 15
                vals = plsc.load_gather(erows_v, [fvec, trow, ccol])
                eg_v[f, pl.ds(j * K + g * 16, 16)] = vals
        pltpu.sync_copy(hrows_v, g0_hbm.at[pl.ds(base, K)])
        return carry

    lax.fori_loop(0, NCHUNK, body, 0)
    pltpu.sync_copy(eg_v, eg_hbm.at[wid])


@functools.partial(
    pl.kernel,
    out_type=(
        jax.ShapeDtypeStruct((E, D_OUT), jnp.float32),
        jax.ShapeDtypeStruct((NW, D_EDGE, EPW), jnp.float32),
    ),
    mesh=_mesh,
    scratch_types=[
        pltpu.VMEM((K,), jnp.int32),
        pltpu.VMEM((K,), jnp.int32),
        pltpu.VMEM((D_EDGE, K), jnp.int32),
        pltpu.VMEM((K, D_OUT), jnp.float32),
        pltpu.VMEM((D_EDGE, K, EAROW), jnp.float32),
        pltpu.VMEM((D_EDGE, EPW), jnp.float32),
        pltpu.SemaphoreType.DMA,
        pltpu.SemaphoreType.DMA,
    ],
    compiler_params=pltpu.CompilerParams(
        use_tc_tiling_on_sc=False, needs_layout_passes=False),
)
def _sc_gather(h_hbm, row_hbm, col_hbm, ea_hbm, g0_hbm, eg_hbm,
               ridx_v, cidx_v, qidx_v, hrows_v, erows_v, eg_v,
               sem_h, sem_e):
    _gather_body(h_hbm, row_hbm, col_hbm, ea_hbm, g0_hbm, eg_hbm,
                 ridx_v, cidx_v, qidx_v, hrows_v, erows_v, eg_v,
                 sem_h, sem_e)


def _mlp_body(g0_ref, eg_ref, w1e_ref, w2_ref, b2_ref, out_ref):
    egb = eg_ref[0]
    ew = lax.dot_general(egb, w1e_ref[...], (((0,), (0,)), ((), ())),
                         preferred_element_type=jnp.float32)
    h1 = jnp.maximum(g0_ref[...] + ew, 0.0)
    h2 = (
        jnp.dot(h1.astype(jnp.bfloat16), w2_ref[...],
                preferred_element_type=jnp.float32)
        + b2_ref[...]
    )
    out_ref[...] = jnp.maximum(h2, 0.0)


HN = 2 * N
HRPT = HN // NS


@functools.partial(
    pl.kernel,
    out_type=jax.ShapeDtypeStruct((NC * HN, 128), jnp.float32),
    mesh=_mesh,
    scratch_types=[
        pltpu.VMEM((K,), jnp.int32),
        pltpu.VMEM((2, K), jnp.int32),
        pltpu.VMEM((2 * K, 128), jnp.float32),
        pltpu.VMEM_SHARED((HN, 128), jnp.float32),
    ],
    compiler_params=pltpu.CompilerParams(
        use_tc_tiling_on_sc=False, needs_layout_passes=False),
)
def _sc_scatter(out2t_hbm, col_hbm, zeros_hbm, part_hbm,
                cidx_v, didx_v, rows_v, acc_sh):
    c = lax.axis_index("c")
    s = lax.axis_index("s")
    wid = s * NC + c
    base0 = wid * EPW
    lane = lax.iota(jnp.int32, 16)
    lane7 = lane & 7
    lhalf = lax.shift_right_logical(lane, 3) & 1

    pltpu.sync_copy(zeros_hbm.at[pl.ds(s * HRPT, HRPT)],
                    acc_sh.at[pl.ds(s * HRPT, HRPT)])
    plsc.subcore_barrier()

    def body(j, carry):
        base = base0 + j * K
        pltpu.sync_copy(col_hbm.at[pl.ds(base, K)], cidx_v)
        pltpu.sync_copy(out2t_hbm.at[pl.ds(2 * base, 2 * K)], rows_v)
        for g in range(2 * K // 16):
            cols = plsc.load_gather(cidx_v, [g * 8 + lane7])
            d = 2 * cols + lhalf
            didx_v[g // 8, pl.ds((g % 8) * 16, 16)] = d
        for q in range(2):
            pltpu.sync_copy(rows_v.at[pl.ds(q * K, K)],
                            acc_sh.at[didx_v.at[q]], add=True)
        return carry

    lax.fori_loop(0, NCHUNK, body, 0)
    plsc.subcore_barrier()

    pltpu.sync_copy(acc_sh.at[pl.ds(s * HRPT, HRPT)],
                    part_hbm.at[pl.ds(c * HN + s * HRPT, HRPT)])


def _combine_body(p_ref, o_ref):
    o_ref[...] = p_ref[0] + p_ref[1]


def kernel(x, edge_index, edge_attr, W1, b1, W2, b2):
    row = edge_index[0]
    col = edge_index[1]
    W1xT = W1[:, :D_IN].T
    W1eT = W1[:, D_IN:].T
    W2T = W2.T
    ea_flat = (edge_attr.reshape(N, N // 128, 128, D_EDGE)
               .transpose(0, 1, 3, 2)
               .reshape(N * N * D_EDGE // EAROW, EAROW))

    H = pl.pallas_call(
        _h_body,
        out_shape=jax.ShapeDtypeStruct((N, D_OUT), jnp.float32),
    )(x, W1xT, b1.reshape(1, D_OUT))

    g0, eg = _sc_gather(H, row, col, ea_flat)

    B = 2048
    BPW = EPW // B
    out2 = pl.pallas_call(
        _mlp_body,
        grid=(E // B,),
        in_specs=[
            pl.BlockSpec((B, D_OUT), lambda i: (i, 0)),
            pl.BlockSpec((1, D_EDGE, B), lambda i: (i // BPW, 0, i % BPW)),
            pl.BlockSpec((D_EDGE, D_OUT), lambda i: (0, 0)),
            pl.BlockSpec((D_OUT, D_OUT), lambda i: (0, 0)),
            pl.BlockSpec((1, D_OUT), lambda i: (0, 0)),
        ],
        out_specs=pl.BlockSpec((B, D_OUT), lambda i: (i, 0)),
        out_shape=jax.ShapeDtypeStruct((E, D_OUT), jnp.float32),
    )(g0, eg, W1eT, W2T.astype(jnp.bfloat16), b2.reshape(1, D_OUT))

    out2t = (out2.reshape(E // 8, 8, 2, 128)
             .transpose(0, 2, 1, 3)
             .reshape(2 * E, 128))
    zeros = jnp.zeros((HN, 128), jnp.float32)
    part = _sc_scatter(out2t, col, zeros)

    BR = 512
    out = pl.pallas_call(
        _combine_body,
        grid=(N // BR,),
        in_specs=[pl.BlockSpec((NC, BR, D_OUT), lambda i: (0, i, 0))],
        out_specs=pl.BlockSpec((BR, D_OUT), lambda i: (i, 0)),
        out_shape=jax.ShapeDtypeStruct((N, D_OUT), jnp.float32),
    )(part.reshape(NC, N, D_OUT))
    return out


# speedup vs baseline: 25.9257x; 1.2297x over previous
import functools

import jax
import jax.numpy as jnp
from jax import lax
from jax.experimental import pallas as pl
from jax.experimental.pallas import tpu as pltpu
from jax.experimental.pallas import tpu_sc as plsc

N = 4096
E = 262144
D_IN = 256
D_EDGE = 4
D_OUT = 256

NC = 2
NS = 16
NW = NC * NS
K = 128
EAROW = 16
HN = 2 * N
HRPT = HN // NS
NHALF = 2

_mesh = plsc.VectorSubcoreMesh(core_axis_name="c", subcore_axis_name="s")


def _h_body(x_ref, w_ref, b_ref, h_ref):
    h_ref[...] = (
        jnp.dot(x_ref[...], w_ref[...], preferred_element_type=jnp.float32)
        + b_ref[...]
    )


@functools.cache
def _make_gather(e_tot):
    epw = e_tot // NW
    nchunk = epw // K

    @functools.partial(
        pl.kernel,
        out_type=(
            jax.ShapeDtypeStruct((e_tot, D_OUT), jnp.float32),
            jax.ShapeDtypeStruct((NW, D_EDGE, epw), jnp.float32),
        ),
        mesh=_mesh,
        scratch_types=[
            pltpu.VMEM((K,), jnp.int32),
            pltpu.VMEM((K,), jnp.int32),
            pltpu.VMEM((D_EDGE, K), jnp.int32),
            pltpu.VMEM((K, D_OUT), jnp.float32),
            pltpu.VMEM((D_EDGE, K, EAROW), jnp.float32),
            pltpu.VMEM((D_EDGE, epw), jnp.float32),
            pltpu.SemaphoreType.DMA,
            pltpu.SemaphoreType.DMA,
        ],
        compiler_params=pltpu.CompilerParams(
            use_tc_tiling_on_sc=False, needs_layout_passes=False),
    )
    def gather(h_hbm, row_hbm, col_hbm, ea_hbm, g0_hbm, eg_hbm,
               ridx_v, cidx_v, qidx_v, hrows_v, erows_v, eg_v, sem_h, sem_e):
        c = lax.axis_index("c")
        s = lax.axis_index("s")
        wid = s * NC + c
        base0 = wid * epw
        lane = lax.iota(jnp.int32, 16)

        def body(j, carry):
            base = base0 + j * K
            pltpu.sync_copy(row_hbm.at[pl.ds(base, K)], ridx_v)
            pltpu.sync_copy(col_hbm.at[pl.ds(base, K)], cidx_v)
            for i in range(K // 16):
                sl = pl.ds(i * 16, 16)
                r = ridx_v[sl]
                cc = cidx_v[sl]
                bq = (r << 10) + ((cc >> 7) << 5) + ((cc >> 4) & 7)
                for f in range(D_EDGE):
                    qidx_v[f, sl] = bq + f * 8
            cp_h = pltpu.async_copy(h_hbm.at[ridx_v], hrows_v, sem_h)
            cps = [pltpu.async_copy(ea_hbm.at[qidx_v.at[f]], erows_v.at[f],
                                    sem_e)
                   for f in range(D_EDGE)]
            cp_h.wait()
            for cp in cps:
                cp.wait()
            for f in range(D_EDGE):
                fvec = jnp.full((16,), f, jnp.int32)
                for g in range(K // 16):
                    sl = pl.ds(g * 16, 16)
                    trow = lane + g * 16
                    ccol = cidx_v[sl] & 15
                    vals = plsc.load_gather(erows_v, [fvec, trow, ccol])
                    eg_v[f, pl.ds(j * K + g * 16, 16)] = vals
            pltpu.sync_copy(hrows_v, g0_hbm.at[pl.ds(base, K)])
            return carry

        lax.fori_loop(0, nchunk, body, 0)
        pltpu.sync_copy(eg_v, eg_hbm.at[wid])

    return gather


def _mlp_body(g0_ref, eg_ref, w1e_ref, w2_ref, b2_ref, out_ref):
    egb = eg_ref[0]
    ew = lax.dot_general(egb, w1e_ref[...], (((0,), (0,)), ((), ())),
                         preferred_element_type=jnp.float32)
    h1 = jnp.maximum(g0_ref[...] + ew, 0.0)
    h2 = (
        jnp.dot(h1.astype(jnp.bfloat16), w2_ref[...],
                preferred_element_type=jnp.float32)
        + b2_ref[...]
    )
    out_ref[...] = jnp.maximum(h2, 0.0)


@functools.cache
def _make_scatter(e_tot):
    epw = e_tot // NW
    nchunk = epw // K

    @functools.partial(
        pl.kernel,
        out_type=jax.ShapeDtypeStruct((NC * HN, 128), jnp.float32),
        mesh=_mesh,
        scratch_types=[
            pltpu.VMEM((K,), jnp.int32),
            pltpu.VMEM((2, K), jnp.int32),
            pltpu.VMEM((2 * K, 128), jnp.float32),
            pltpu.VMEM_SHARED((HN, 128), jnp.float32),
        ],
        compiler_params=pltpu.CompilerParams(
            use_tc_tiling_on_sc=False, needs_layout_passes=False),
    )
    def scatter(out2t_hbm, col_hbm, zeros_hbm, part_hbm,
                cidx_v, didx_v, rows_v, acc_sh):
        c = lax.axis_index("c")
        s = lax.axis_index("s")
        wid = s * NC + c
        base0 = wid * epw
        lane = lax.iota(jnp.int32, 16)
        lane7 = lane & 7
        lhalf = lax.shift_right_logical(lane, 3) & 1

        pltpu.sync_copy(zeros_hbm.at[pl.ds(s * HRPT, HRPT)],
                        acc_sh.at[pl.ds(s * HRPT, HRPT)])
        plsc.subcore_barrier()

        def body(j, carry):
            base = base0 + j * K
            pltpu.sync_copy(col_hbm.at[pl.ds(base, K)], cidx_v)
            pltpu.sync_copy(out2t_hbm.at[pl.ds(2 * base, 2 * K)], rows_v)
            for g in range(2 * K // 16):
                cols = plsc.load_gather(cidx_v, [g * 8 + lane7])
                d = 2 * cols + lhalf
                didx_v[g // 8, pl.ds((g % 8) * 16, 16)] = d
            for q in range(2):
                pltpu.sync_copy(rows_v.at[pl.ds(q * K, K)],
                                acc_sh.at[didx_v.at[q]], add=True)
            return carry

        lax.fori_loop(0, nchunk, body, 0)
        plsc.subcore_barrier()

        pltpu.sync_copy(acc_sh.at[pl.ds(s * HRPT, HRPT)],
                        part_hbm.at[pl.ds(c * HN + s * HRPT, HRPT)])

    return scatter


def _combine_body(p0_ref, p1_ref, o_ref):
    o_ref[...] = (p0_ref[0] + p0_ref[1]) + (p1_ref[0] + p1_ref[1])


def kernel(x, edge_index, edge_attr, W1, b1, W2, b2):
    row = edge_index[0]
    col = edge_index[1]
    W1eT = W1[:, D_IN:].T
    W2Tb = W2.T.astype(jnp.bfloat16)
    ea_flat = (edge_attr.reshape(N, N // 128, 128, D_EDGE)
               .transpose(0, 1, 3, 2)
               .reshape(N * N * D_EDGE // EAROW, EAROW))

    H = pl.pallas_call(
        _h_body,
        out_shape=jax.ShapeDtypeStruct((N, D_OUT), jnp.float32),
    )(x, W1[:, :D_IN].T, b1.reshape(1, D_OUT))

    EH = E // NHALF
    gather = _make_gather(EH)
    scatter = _make_scatter(EH)
    zeros = jnp.zeros((HN, 128), jnp.float32)

    B = 2048
    BPW = max(1, (EH // NW) // B)

    parts = []
    for h in range(NHALF):
        row_h = lax.slice_in_dim(row, h * EH, (h + 1) * EH)
        col_h = lax.slice_in_dim(col, h * EH, (h + 1) * EH)

        g0, eg = gather(H, row_h, col_h, ea_flat)

        out2 = pl.pallas_call(
            _mlp_body,
            grid=(EH // B,),
            in_specs=[
                pl.BlockSpec((B, D_OUT), lambda i: (i, 0)),
                pl.BlockSpec((1, D_EDGE, B),
                             lambda i, bpw=BPW: (i // bpw, 0, i % bpw)),
                pl.BlockSpec((D_EDGE, D_OUT), lambda i: (0, 0)),
                pl.BlockSpec((D_OUT, D_OUT), lambda i: (0, 0)),
                pl.BlockSpec((1, D_OUT), lambda i: (0, 0)),
            ],
            out_specs=pl.BlockSpec((B, D_OUT), lambda i: (i, 0)),
            out_shape=jax.ShapeDtypeStruct((EH, D_OUT), jnp.float32),
        )(g0, eg, W1eT, W2Tb, b2.reshape(1, D_OUT))

        out2t = (out2.reshape(EH // 8, 8, 2, 128)
                 .transpose(0, 2, 1, 3)
                 .reshape(2 * EH, 128))
        parts.append(scatter(out2t, col_h, zeros))

    BR = 512
    out = pl.pallas_call(
        _combine_body,
        grid=(N // BR,),
        in_specs=[pl.BlockSpec((NC, BR, D_OUT), lambda i: (0, i, 0)),
                  pl.BlockSpec((NC, BR, D_OUT), lambda i: (0, i, 0))],
        out_specs=pl.BlockSpec((BR, D_OUT), lambda i: (i, 0)),
        out_shape=jax.ShapeDtypeStruct((N, D_OUT), jnp.float32),
    )(parts[0].reshape(NC, N, D_OUT), parts[1].reshape(NC, N, D_OUT))
    return out
